# baseline (speedup 1.0000x reference)
import functools

import jax
import jax.numpy as jnp
from jax import lax
from jax.experimental import pallas as pl
from jax.experimental.pallas import tpu as pltpu
from jax.experimental.pallas import tpu_sc as plsc

_B = 16384
_D = 16
_V = 1000000
_G = 128 // _D
_VG = _V // _G

_NC = 2
_NS = 16
_NW = _NC * _NS
_BPW = _B // _NW
_GCH = 128
_NGCH = _BPW // _GCH


def _sc_gather6(gu_idx, gp_idx, gn_idx, t_gu, t_gi, t_mu, t_mi):
    mesh = plsc.VectorSubcoreMesh(core_axis_name="c", subcore_axis_name="s")
    out_t = tuple(jax.ShapeDtypeStruct((_B, 128), jnp.float32)
                  for _ in range(6))
    scratch = (
        [pltpu.VMEM((_BPW,), jnp.int32) for _ in range(3)]
        + [pltpu.VMEM((_GCH, 128), jnp.float32) for _ in range(6)]
        + [pltpu.SemaphoreType.DMA, pltpu.SemaphoreType.DMA]
    )

    @functools.partial(pl.kernel, mesh=mesh, out_type=out_t,
                       scratch_types=scratch)
    def body(u_h, p_h, n_h, tgu_h, tgi_h, tmu_h, tmi_h,
             o_gu, o_gp, o_gn, o_mu, o_mp, o_mn,
             uv, pv, nv, b0, b1, b2, b3, b4, b5, sem, wsem):
        wid = lax.axis_index("s") * _NC + lax.axis_index("c")
        base = wid * _BPW
        pltpu.sync_copy(u_h.at[pl.ds(base, _BPW)], uv)
        pltpu.sync_copy(p_h.at[pl.ds(base, _BPW)], pv)
        pltpu.sync_copy(n_h.at[pl.ds(base, _BPW)], nv)
        bufs = (b0, b1, b2, b3, b4, b5)
        jobs = ((tgu_h, uv, o_gu), (tgi_h, pv, o_gp), (tgi_h, nv, o_gn),
                (tmu_h, uv, o_mu), (tmi_h, pv, o_mp), (tmi_h, nv, o_mn))
        writes = []
        for j in range(_NGCH):
            sl = pl.ds(j * _GCH, _GCH)
            gathers = [
                pltpu.async_copy(tab.at[iv.at[sl]], bufs[k], sem)
                for k, (tab, iv, _) in enumerate(jobs)
            ]
            for w in writes:
                w.wait()
            for g in gathers:
                g.wait()
            writes = [
                pltpu.async_copy(bufs[k], out.at[pl.ds(base + j * _GCH,
                                                       _GCH)], wsem)
                for k, (_, _, out) in enumerate(jobs)
            ]
        for w in writes:
            w.wait()

    return body(gu_idx, gp_idx, gn_idx, t_gu, t_gi, t_mu, t_mi)


_BLK = 2048


def _r16(x):
    return x.astype(jnp.bfloat16).astype(jnp.float32)


def _pick16(x128, off):
    out = jnp.zeros((x128.shape[0], _D), jnp.float32)
    for o in range(_G):
        out = jnp.where(off == o, x128[:, o * _D:(o + 1) * _D], out)
    return out


def _tc_body(gu_r, gp_r, gn_r, mu_r, mp_r, mn_r, ou_r, op_r, on_r,
             w1_r, b1_r, w2_r, b2_r, w3_r, b3_r, wdg_r, wdm_r, bd_r, out_r):
    f32 = jnp.float32
    hi = lax.Precision.HIGHEST
    ou = ou_r[...]
    op = op_r[...]
    on = on_r[...]
    gu = _pick16(gu_r[...], ou)
    gmf_p = jax.nn.sigmoid(gu * _pick16(gp_r[...], op))
    gmf_n = jax.nn.sigmoid(gu * _pick16(gn_r[...], on))

    w1 = _r16(w1_r[...])
    w1a, w1b = w1[:_D], w1[_D:]
    b1 = b1_r[...]
    w2 = _r16(w2_r[...])
    b2 = b2_r[...]
    w3 = _r16(w3_r[...])
    b3 = b3_r[...]
    mu = _r16(_pick16(mu_r[...], ou))
    u_part = jnp.dot(mu, w1a, preferred_element_type=f32, precision=hi)

    def dnn(xi):
        h = u_part + jnp.dot(_r16(xi), w1b, preferred_element_type=f32,
                             precision=hi) + b1
        h = jnp.maximum(h, 0.0)
        h = jnp.maximum(jnp.dot(_r16(h), w2, preferred_element_type=f32,
                                precision=hi) + b2, 0.0)
        h = jnp.maximum(jnp.dot(_r16(h), w3, preferred_element_type=f32,
                                precision=hi) + b3, 0.0)
        return h

    hp = dnn(_pick16(mp_r[...], op))
    hn = dnn(_pick16(mn_r[...], on))

    wdg = _r16(wdg_r[...])
    wdm = _r16(wdm_r[...])
    bd = bd_r[...]
    pos = (jnp.sum(_r16(gmf_p) * wdg, axis=1, keepdims=True)
           + jnp.sum(_r16(hp) * wdm, axis=1, keepdims=True) + bd)
    neg = (jnp.sum(_r16(gmf_n) * wdg, axis=1, keepdims=True)
           + jnp.sum(_r16(hn) * wdm, axis=1, keepdims=True) + bd)
    out_r[...] = jnp.concatenate([pos, neg], axis=1)


def _tc_mlp(gu, gp, gn, mu, mp_, mn, ou, op, on,
            w1, b1, w2, b2, w3, b3, wd, bd):
    grid = (_B // _BLK,)
    row_spec = pl.BlockSpec((_BLK, 128), lambda i: (i, 0))
    off_spec = pl.BlockSpec((_BLK, 1), lambda i: (i, 0))
    full = lambda s: pl.BlockSpec(s, lambda i: (0, 0))
    return pl.pallas_call(
        _tc_body,
        grid=grid,
        in_specs=[row_spec] * 6 + [off_spec] * 3 + [
            full((2 * _D, 64)), full((1, 64)),
            full((64, 16)), full((1, 16)),
            full((16, 8)), full((1, 8)),
            full((1, _D)), full((1, 8)), full((1, 1)),
        ],
        out_specs=pl.BlockSpec((_BLK, 2), lambda i: (i, 0)),
        out_shape=jax.ShapeDtypeStruct((_B, 2), jnp.float32),
    )(gu, gp, gn, mu, mp_, mn, ou, op, on,
      w1, b1.reshape(1, 64), w2, b2.reshape(1, 16), w3, b3.reshape(1, 8),
      wd[:_D].reshape(1, _D), wd[_D:].reshape(1, 8), bd.reshape(1, 1))


def kernel(user_inputs, pos_inputs, neg_inputs,
           gmf_user_table, gmf_item_table, mlp_user_table, mlp_item_table,
           w1, b1, w2, b2, w3, b3, wd, bd):
    u = user_inputs.reshape(_B).astype(jnp.int32)
    p = pos_inputs.reshape(_B).astype(jnp.int32)
    n = neg_inputs.reshape(_B).astype(jnp.int32)
    tabs = [t.reshape(_VG, 128) for t in (gmf_user_table, gmf_item_table,
                                          mlp_user_table, mlp_item_table)]
    gu, gp, gn, mu, mp_, mn = _sc_gather6(
        u >> 3, p >> 3, n >> 3, *tabs)
    return _tc_mlp(gu, gp, gn, mu, mp_, mn,
                   (u & 7).reshape(_B, 1), (p & 7).reshape(_B, 1),
                   (n & 7).reshape(_B, 1),
                   w1, b1, w2, b2, w3, b3, wd, bd)


# speedup vs baseline: 1.1265x; 1.1265x over previous
import functools

import jax
import jax.numpy as jnp
from jax import lax
from jax.experimental import pallas as pl
from jax.experimental.pallas import tpu as pltpu
from jax.experimental.pallas import tpu_sc as plsc

_B = 16384
_D = 16
_V = 1000000
_PK = 128 // _D

_NC = 2
_NS = 16
_NW = _NC * _NS
_BPW = _B // _NW
_GCH = 128
_NGCH = _BPW // _GCH


def _sc_gather6(u8, p8, n8, t_gu, t_gi, t_mu, t_mi):
    mesh = plsc.VectorSubcoreMesh(core_axis_name="c", subcore_axis_name="s")
    out_t = tuple(jax.ShapeDtypeStruct((_B // _PK, 128), jnp.float32)
                  for _ in range(6))
    scratch = (
        [pltpu.VMEM((_BPW,), jnp.int32) for _ in range(3)]
        + [pltpu.VMEM((_GCH, _D), jnp.float32) for _ in range(6)]
        + [pltpu.VMEM((_GCH // _PK, 128), jnp.float32) for _ in range(6)]
        + [pltpu.SemaphoreType.DMA, pltpu.SemaphoreType.DMA]
    )

    @functools.partial(
        pl.kernel, mesh=mesh, out_type=out_t, scratch_types=scratch,
        compiler_params=pltpu.CompilerParams(use_tc_tiling_on_sc=False))
    def body(u_h, p_h, n_h, tgu_h, tgi_h, tmu_h, tmi_h,
             o_gu, o_gp, o_gn, o_mu, o_mp, o_mn,
             uv, pv, nv, b0, b1, b2, b3, b4, b5,
             w0, w1, w2, w3, w4, w5, sem, wsem):
        wid = lax.axis_index("s") * _NC + lax.axis_index("c")
        base = wid * _BPW
        pltpu.sync_copy(u_h.at[pl.ds(base, _BPW)], uv)
        pltpu.sync_copy(p_h.at[pl.ds(base, _BPW)], pv)
        pltpu.sync_copy(n_h.at[pl.ds(base, _BPW)], nv)
        bufs = (b0, b1, b2, b3, b4, b5)
        wbs = (w0, w1, w2, w3, w4, w5)
        jobs = ((tgu_h, uv, o_gu), (tgi_h, pv, o_gp), (tgi_h, nv, o_gn),
                (tmu_h, uv, o_mu), (tmi_h, pv, o_mp), (tmi_h, nv, o_mn))
        writes = []
        for j in range(_NGCH):
            sl = pl.ds(j * _GCH, _GCH)
            gathers = [
                pltpu.async_copy(tab.at[iv.at[sl]], bufs[k], sem)
                for k, (tab, iv, _) in enumerate(jobs)
            ]
            for w in writes:
                w.wait()
            for g in gathers:
                g.wait()
            for k in range(6):
                for r in range(_GCH):
                    wbs[k][r // _PK, pl.ds((r % _PK) * _D, _D)] = (
                        bufs[k][r, :])
            writes = [
                pltpu.async_copy(
                    wbs[k],
                    out.at[pl.ds(base // _PK + j * (_GCH // _PK),
                                 _GCH // _PK)],
                    wsem)
                for k, (_, _, out) in enumerate(jobs)
            ]
        for w in writes:
            w.wait()

    return body(u8, p8, n8, t_gu, t_gi, t_mu, t_mi)


_BLKP = 256


def _r16(x):
    return x.astype(jnp.bfloat16).astype(jnp.float32)


def _tc_body(gu_r, gp_r, gn_r, mu_r, mp_r, mn_r,
             w1a_r, w1b_r, b1_r, w2_r, b2_r, w3_r, b3_r,
             wdg_r, wdm_r, bd_r, pos_r, neg_r):
    f32 = jnp.float32
    hi = lax.Precision.HIGHEST
    gu = gu_r[...]
    gmf_p = jax.nn.sigmoid(gu * gp_r[...])
    gmf_n = jax.nn.sigmoid(gu * gn_r[...])

    w1a = _r16(w1a_r[...])
    w1b = _r16(w1b_r[...])
    b1 = b1_r[...]
    w2 = _r16(w2_r[...])
    b2 = b2_r[...]
    w3 = _r16(w3_r[...])
    b3 = b3_r[...]
    u_part = jnp.dot(_r16(mu_r[...]), w1a, preferred_element_type=f32,
                     precision=hi)

    def dnn(xi):
        h = u_part + jnp.dot(_r16(xi), w1b, preferred_element_type=f32,
                             precision=hi) + b1
        h = jnp.maximum(h, 0.0)
        h = jnp.maximum(jnp.dot(_r16(h), w2, preferred_element_type=f32,
                                precision=hi) + b2, 0.0)
        h = jnp.maximum(jnp.dot(_r16(h), w3, preferred_element_type=f32,
                                precision=hi) + b3, 0.0)
        return h

    hp = dnn(mp_r[...])
    hn = dnn(mn_r[...])

    wdg = _r16(wdg_r[...])
    wdm = _r16(wdm_r[...])
    bd = bd_r[0, 0]
    qgp = _r16(gmf_p) * wdg
    qgn = _r16(gmf_n) * wdg
    qmp = _r16(hp) * wdm
    qmn = _r16(hn) * wdm
    ent = lax.broadcasted_iota(jnp.int32, (_BLKP, _PK), 1)
    pos = jnp.zeros((_BLKP, _PK), f32)
    neg = jnp.zeros((_BLKP, _PK), f32)
    for k in range(_PK):
        sp = (jnp.sum(qgp[:, k * _D:(k + 1) * _D], axis=1, keepdims=True)
              + jnp.sum(qmp[:, k * 8:(k + 1) * 8], axis=1, keepdims=True)
              + bd)
        sn = (jnp.sum(qgn[:, k * _D:(k + 1) * _D], axis=1, keepdims=True)
              + jnp.sum(qmn[:, k * 8:(k + 1) * 8], axis=1, keepdims=True)
              + bd)
        pos = jnp.where(ent == k, sp, pos)
        neg = jnp.where(ent == k, sn, neg)
    pos_r[...] = pos
    neg_r[...] = neg


def _tc_mlp(gu, gp, gn, mu, mp_, mn, w1, b1, w2, b2, w3, b3, wd, bd):
    bd8 = lambda m: jax.scipy.linalg.block_diag(*([m] * _PK))
    tile = lambda v: jnp.tile(v, _PK)
    grid = (_B // _PK // _BLKP,)
    row_spec = pl.BlockSpec((_BLKP, 128), lambda i: (i, 0))
    full = lambda s: pl.BlockSpec(s, lambda i: (0, 0))
    pos, neg = pl.pallas_call(
        _tc_body,
        grid=grid,
        in_specs=[row_spec] * 6 + [
            full((128, 512)), full((128, 512)), full((1, 512)),
            full((512, 128)), full((1, 128)),
            full((128, 64)), full((1, 64)),
            full((1, 128)), full((1, 64)), full((1, 1)),
        ],
        out_specs=[pl.BlockSpec((_BLKP, _PK), lambda i: (i, 0))] * 2,
        out_shape=[jax.ShapeDtypeStruct((_B // _PK, _PK), jnp.float32)] * 2,
    )(gu, gp, gn, mu, mp_, mn,
      bd8(w1[:_D]), bd8(w1[_D:]), tile(b1).reshape(1, 512),
      bd8(w2), tile(b2).reshape(1, 128),
      bd8(w3), tile(b3).reshape(1, 64),
      tile(wd[:_D, 0]).reshape(1, 128), tile(wd[_D:, 0]).reshape(1, 64),
      bd.reshape(1, 1))
    return jnp.stack([pos.reshape(_B), neg.reshape(_B)], axis=-1)


def kernel(user_inputs, pos_inputs, neg_inputs,
           gmf_user_table, gmf_item_table, mlp_user_table, mlp_item_table,
           w1, b1, w2, b2, w3, b3, wd, bd):
    u = user_inputs.reshape(_B).astype(jnp.int32) << 3
    p = pos_inputs.reshape(_B).astype(jnp.int32) << 3
    n = neg_inputs.reshape(_B).astype(jnp.int32) << 3
    gu, gp, gn, mu, mp_, mn = _sc_gather6(
        u, p, n, gmf_user_table, gmf_item_table,
        mlp_user_table, mlp_item_table)
    return _tc_mlp(gu, gp, gn, mu, mp_, mn,
                   w1, b1, w2, b2, w3, b3, wd, bd)
